# Initial kernel scaffold; baseline (speedup 1.0000x reference)
import jax
import jax.numpy as jnp
from jax.experimental import pallas as pl


def kernel(edge_index, embedding, W1, b1, W2, b2, W3, b3, W4, b4):
    raise NotImplementedError("write your pallas kernel here")


# speedup vs baseline: 7.3118x; 7.3118x over previous
import functools

import jax
import jax.numpy as jnp
from jax import lax
from jax.experimental import pallas as pl
from jax.experimental.pallas import tpu as pltpu
from jax.experimental.pallas import tpu_sc as plsc

N = 100000
E = 1600000
BN = 512
NPAD = 100352
GRID = NPAD // BN
RPT = NPAD // 16
NQ = 16
RQ = RPT // NQ
EPAD = 1638400
IDXROWS = EPAD // 128
K = 8

_MESH = plsc.VectorSubcoreMesh(core_axis_name="c", subcore_axis_name="s")
_SC_PARAMS = pltpu.CompilerParams(use_tc_tiling_on_sc=False)


def _zero_init(ref, nrows):
    def body(i, carry):
        ref[i, :] = jnp.zeros((16,), jnp.float32)
        return carry
    lax.fori_loop(0, nrows, body, 0)


def _make_deg_kernel():
    @functools.partial(
        pl.kernel,
        out_type=jax.ShapeDtypeStruct((2, NPAD, 16), jnp.float32),
        mesh=_MESH,
        compiler_params=_SC_PARAMS,
        scratch_types=[
            pltpu.VMEM_SHARED((NPAD, 16), jnp.float32),
            pltpu.VMEM((K, 128), jnp.int32),
            pltpu.VMEM((128, 16), jnp.float32),
            pltpu.VMEM((RQ, 16), jnp.float32),
            pltpu.SemaphoreType.DMA,
        ],
    )
    def deg_kernel(dst_hbm, deg_hbm, deg_sp, dst_v, ones_v, zero_v, sem):
        core = lax.axis_index("c")
        sub = lax.axis_index("s")
        _zero_init(zero_v, RQ)

        def ones_body(i, carry):
            ones_v[i, :] = jnp.ones((16,), jnp.float32)
            return carry
        lax.fori_loop(0, 128, ones_body, 0)

        zrow = sub * RPT
        for q in range(NQ):
            pltpu.sync_copy(zero_v, deg_sp.at[pl.ds(zrow + q * RQ, RQ)])
        plsc.subcore_barrier()

        row0 = (sub * 2 + core) * (IDXROWS // 32)

        def body(it, carry):
            pltpu.sync_copy(dst_hbm.at[pl.ds(row0 + it * K, K)], dst_v)
            for j in range(K):
                pltpu.sync_copy(ones_v, deg_sp.at[dst_v.at[j]], add=True)
            return carry
        lax.fori_loop(0, (IDXROWS // 32) // K, body, 0)
        plsc.subcore_barrier()

        for q in range(NQ):
            pltpu.sync_copy(deg_sp.at[pl.ds(zrow + q * RQ, RQ)],
                            deg_hbm.at[core].at[pl.ds(zrow + q * RQ, RQ)])

    return deg_kernel


def _make_scatter_kernel(C):
    n_outer = (IDXROWS // 16) // K

    @functools.partial(
        pl.kernel,
        out_type=jax.ShapeDtypeStruct((C, NPAD, 16), jnp.float32),
        mesh=_MESH,
        compiler_params=_SC_PARAMS,
        scratch_types=[
            pltpu.VMEM_SHARED((NPAD, 16), jnp.float32),
            pltpu.VMEM((K, 128), jnp.int32),
            pltpu.VMEM((K, 128), jnp.int32),
            pltpu.VMEM((K, 128, 16), jnp.float32),
            pltpu.VMEM((RQ, 16), jnp.float32),
            pltpu.SemaphoreType.DMA,
        ],
    )
    def scatter_kernel(hs_hbm, src_hbm, dst_hbm, acc_hbm,
                       acc_sp, src_v, dst_v, rows_v, zero_v, sem):
        core = lax.axis_index("c")
        sub = lax.axis_index("s")
        _zero_init(zero_v, RQ)
        zrow = sub * RPT
        row0 = sub * (IDXROWS // 16)

        for ci in range(C // 2):
            chunk = ci * 2 + core
            for q in range(NQ):
                pltpu.sync_copy(zero_v, acc_sp.at[pl.ds(zrow + q * RQ, RQ)])
            plsc.subcore_barrier()

            tab = hs_hbm.at[chunk]

            def body(it, carry):
                base = row0 + it * K
                pltpu.sync_copy(src_hbm.at[pl.ds(base, K)], src_v)
                pltpu.sync_copy(dst_hbm.at[pl.ds(base, K)], dst_v)
                descs = [pltpu.async_copy(tab.at[src_v.at[j]], rows_v.at[j], sem)
                         for j in range(K)]
                for d in descs:
                    d.wait()
                for j in range(K):
                    pltpu.sync_copy(rows_v.at[j], acc_sp.at[dst_v.at[j]], add=True)
                return carry
            lax.fori_loop(0, n_outer, body, 0)
            plsc.subcore_barrier()

            for q in range(NQ):
                pltpu.sync_copy(acc_sp.at[pl.ds(zrow + q * RQ, RQ)],
                                acc_hbm.at[chunk].at[pl.ds(zrow + q * RQ, RQ)])
            plsc.subcore_barrier()

    return scatter_kernel


_deg_call = _make_deg_kernel()
_scatter2 = _make_scatter_kernel(2)
_scatter4 = _make_scatter_kernel(4)
_scatter8 = _make_scatter_kernel(8)


def _chunk_spec(C):
    return pl.BlockSpec((C, BN, 16), lambda i: (0, i, 0))


def _row_spec(D):
    return pl.BlockSpec((BN, D), lambda i: (i, 0))


def _full_spec(a, b):
    return pl.BlockSpec((a, b), lambda i: (0, 0))


def _tc_head(emb_ref, deg_ref, w_ref, dis_ref, hs_ref):
    dis = lax.rsqrt(1.0 + deg_ref[0][:, :1] + deg_ref[1][:, :1])
    dis_ref[...] = dis
    h = jnp.dot(emb_ref[...], w_ref[...],
                preferred_element_type=jnp.float32) * dis
    for c in range(hs_ref.shape[0]):
        hs_ref[c] = h[:, c * 16:(c + 1) * 16]


def _tc_mid(acc_ref, hs_ref, dis_ref, b_ref, w_ref, *out_refs):
    C_in = acc_ref.shape[0]
    x = jnp.concatenate([acc_ref[c] + hs_ref[c] for c in range(C_in)], axis=1)
    dis = dis_ref[...]
    out = jnp.maximum(x * dis + b_ref[...], 0.0)
    hsn_ref = out_refs[-1]
    if len(out_refs) == 2:
        out_refs[0][...] = out
    hn = jnp.dot(out, w_ref[...], preferred_element_type=jnp.float32) * dis
    for c in range(hsn_ref.shape[0]):
        hsn_ref[c] = hn[:, c * 16:(c + 1) * 16]


def _tc_tail(acc_ref, hs_ref, dis_ref, b_ref, recon_ref):
    C_in = acc_ref.shape[0]
    x = jnp.concatenate([acc_ref[c] + hs_ref[c] for c in range(C_in)], axis=1)
    recon_ref[...] = x * dis_ref[...] + b_ref[...]


def kernel(edge_index, embedding, W1, b1, W2, b2, W3, b3, W4, b4):
    src = edge_index[0]
    dst = edge_index[1]
    src2d = jnp.concatenate(
        [src, jnp.zeros((EPAD - E,), jnp.int32)]).reshape(IDXROWS, 128)
    dst2d = jnp.concatenate(
        [dst, jnp.full((EPAD - E,), N, jnp.int32)]).reshape(IDXROWS, 128)
    emb_p = jnp.concatenate(
        [embedding, jnp.zeros((NPAD - N, embedding.shape[1]), jnp.float32)])

    degs = _deg_call(dst2d)

    dis, hs1 = pl.pallas_call(
        _tc_head,
        grid=(GRID,),
        in_specs=[_row_spec(32),
                  pl.BlockSpec((2, BN, 16), lambda i: (0, i, 0)),
                  _full_spec(32, 64)],
        out_specs=[_row_spec(1), _chunk_spec(4)],
        out_shape=[jax.ShapeDtypeStruct((NPAD, 1), jnp.float32),
                   jax.ShapeDtypeStruct((4, NPAD, 16), jnp.float32)],
    )(emb_p, degs, W1)

    acc1 = _scatter4(hs1, src2d, dst2d)

    hs2 = pl.pallas_call(
        _tc_mid,
        grid=(GRID,),
        in_specs=[_chunk_spec(4), _chunk_spec(4), _row_spec(1),
                  _full_spec(1, 64), _full_spec(64, 32)],
        out_specs=[_chunk_spec(2)],
        out_shape=[jax.ShapeDtypeStruct((2, NPAD, 16), jnp.float32)],
    )(acc1, hs1, dis, b1.reshape(1, 64), W2)[0]

    acc2 = _scatter2(hs2, src2d, dst2d)

    z_p, hs3 = pl.pallas_call(
        _tc_mid,
        grid=(GRID,),
        in_specs=[_chunk_spec(2), _chunk_spec(2), _row_spec(1),
                  _full_spec(1, 32), _full_spec(32, 64)],
        out_specs=[_row_spec(32), _chunk_spec(4)],
        out_shape=[jax.ShapeDtypeStruct((NPAD, 32), jnp.float32),
                   jax.ShapeDtypeStruct((4, NPAD, 16), jnp.float32)],
    )(acc2, hs2, dis, b2.reshape(1, 32), W3)

    acc3 = _scatter4(hs3, src2d, dst2d)

    hs4 = pl.pallas_call(
        _tc_mid,
        grid=(GRID,),
        in_specs=[_chunk_spec(4), _chunk_spec(4), _row_spec(1),
                  _full_spec(1, 64), _full_spec(64, 128)],
        out_specs=[_chunk_spec(8)],
        out_shape=[jax.ShapeDtypeStruct((8, NPAD, 16), jnp.float32)],
    )(acc3, hs3, dis, b3.reshape(1, 64), W4)[0]

    acc4 = _scatter8(hs4, src2d, dst2d)

    recon_p = pl.pallas_call(
        _tc_tail,
        grid=(GRID,),
        in_specs=[_chunk_spec(8), _chunk_spec(8), _row_spec(1),
                  _full_spec(1, 128)],
        out_specs=pl.BlockSpec((BN, 128), lambda i: (i, 0)),
        out_shape=jax.ShapeDtypeStruct((NPAD, 128), jnp.float32),
    )(acc4, hs4, dis, b4.reshape(1, 128))

    return (recon_p[:N], z_p[:N])
